# baseline (speedup 1.0000x reference)
import functools

import jax
import jax.numpy as jnp
from jax import lax
from jax.experimental import pallas as pl
from jax.experimental.pallas import tpu as pltpu
from jax.experimental.pallas import tpu_sc as plsc

L = 16
NC, NS = 2, 16
NW = NC * NS
CH = 128


def _sqrt16(x):
    xs = jnp.maximum(x, jnp.float32(1e-30))
    i = lax.bitcast_convert_type(xs, jnp.int32)
    i = jnp.int32(0x5F3759DF) - lax.shift_right_logical(i, jnp.int32(1))
    y = lax.bitcast_convert_type(i, jnp.float32)
    xh = xs * jnp.float32(0.5)
    for _ in range(3):
        y = y * (jnp.float32(1.5) - xh * y * y)
    return xs * y


def kernel(heads, relations, tails, entity_emb, relation_emb):
    B = heads.shape[0]
    D = entity_emb.shape[1]
    b_per_w = B // NW
    n_chunks = b_per_w // CH
    mesh = plsc.VectorSubcoreMesh(core_axis_name="c", subcore_axis_name="s")

    @functools.partial(
        pl.kernel,
        mesh=mesh,
        compiler_params=pltpu.CompilerParams(
            needs_layout_passes=False, use_tc_tiling_on_sc=False),
        out_type=jax.ShapeDtypeStruct((B,), jnp.float32),
        scratch_types=[
            pltpu.VMEM((b_per_w,), jnp.int32),
            pltpu.VMEM((b_per_w,), jnp.int32),
            pltpu.VMEM((b_per_w,), jnp.int32),
            pltpu.VMEM((CH, D), jnp.float32),
            pltpu.VMEM((CH, D), jnp.float32),
            pltpu.VMEM((CH, D), jnp.float32),
            pltpu.VMEM((CH,), jnp.float32),
            pltpu.SemaphoreType.DMA,
            pltpu.SemaphoreType.DMA,
            pltpu.SemaphoreType.DMA,
        ],
    )
    def sc_kernel(heads_hbm, rels_hbm, tails_hbm, ent_hbm, rel_hbm, out_hbm,
                  hidx, ridx, tidx, hrows, rrows, trows, outc,
                  sem_h, sem_r, sem_t):
        wid = lax.axis_index("s") * NC + lax.axis_index("c")
        base = wid * b_per_w
        pltpu.sync_copy(heads_hbm.at[pl.ds(base, b_per_w)], hidx)
        pltpu.sync_copy(rels_hbm.at[pl.ds(base, b_per_w)], ridx)
        pltpu.sync_copy(tails_hbm.at[pl.ds(base, b_per_w)], tidx)

        @pl.loop(0, n_chunks)
        def _chunk(ci):
            off = ci * CH
            cp_h = pltpu.async_copy(
                ent_hbm.at[hidx.at[pl.ds(off, CH)]], hrows, sem_h)
            cp_r = pltpu.async_copy(
                rel_hbm.at[ridx.at[pl.ds(off, CH)]], rrows, sem_r)
            cp_t = pltpu.async_copy(
                ent_hbm.at[tidx.at[pl.ds(off, CH)]], trows, sem_t)
            cp_h.wait()
            cp_r.wait()
            cp_t.wait()

            lane = lax.iota(jnp.int32, L)

            @pl.loop(0, CH // L)
            def _grp(g):
                def row_body(k, vec):
                    w = g * L + k
                    acc = None
                    for c in range(D // L):
                        sl = pl.ds(c * L, L)
                        d = hrows[w, sl] + rrows[w, sl] - trows[w, sl]
                        sq = d * d
                        acc = sq if acc is None else acc + sq
                    s = jnp.sum(acc)
                    return jnp.where(lane == k, s, vec)

                vec = lax.fori_loop(0, L, row_body,
                                    jnp.zeros((L,), jnp.float32))
                outc[pl.ds(g * L, L)] = _sqrt16(vec)

            pltpu.sync_copy(outc, out_hbm.at[pl.ds(base + off, CH)])

    return sc_kernel(heads, relations, tails, entity_emb, relation_emb)


# speedup vs baseline: 2.0802x; 2.0802x over previous
import functools

import jax
import jax.numpy as jnp
from jax import lax
from jax.experimental import pallas as pl
from jax.experimental.pallas import tpu as pltpu
from jax.experimental.pallas import tpu_sc as plsc

L = 16
NC, NS = 2, 16
NW = NC * NS
CH = 32


def _sqrt16(x):
    xs = jnp.maximum(x, jnp.float32(1e-30))
    i = lax.bitcast_convert_type(xs, jnp.int32)
    i = jnp.int32(0x5F3759DF) - lax.shift_right_logical(i, jnp.int32(1))
    y = lax.bitcast_convert_type(i, jnp.float32)
    xh = xs * jnp.float32(0.5)
    for _ in range(3):
        y = y * (jnp.float32(1.5) - xh * y * y)
    return xs * y


def kernel(heads, relations, tails, entity_emb, relation_emb):
    B = heads.shape[0]
    N, D = entity_emb.shape
    b_per_w = B // NW
    n_chunks = b_per_w // CH
    ent3 = entity_emb.reshape(N // 8, 8, D)
    relp = jnp.pad(relation_emb, ((0, 0), (0, 128 - D)))
    mesh = plsc.VectorSubcoreMesh(core_axis_name="c", subcore_axis_name="s")

    @functools.partial(
        pl.kernel,
        mesh=mesh,
        compiler_params=pltpu.CompilerParams(needs_layout_passes=False),
        out_type=jax.ShapeDtypeStruct((B,), jnp.float32),
        scratch_types=[
            pltpu.VMEM((b_per_w,), jnp.int32),
            pltpu.VMEM((b_per_w,), jnp.int32),
            pltpu.VMEM((b_per_w,), jnp.int32),
            pltpu.VMEM((CH, 8, D), jnp.float32),
            pltpu.VMEM((CH, 8, D), jnp.float32),
            pltpu.VMEM((CH, 128), jnp.float32),
            pltpu.VMEM((CH,), jnp.float32),
            pltpu.SemaphoreType.DMA,
            pltpu.SemaphoreType.DMA,
            pltpu.SemaphoreType.DMA,
        ],
    )
    def sc_kernel(heads_hbm, rels_hbm, tails_hbm, ent_hbm, rel_hbm, out_hbm,
                  hidx, tidx, ridx, hslab, tslab, rrow, outc,
                  sem_h, sem_r, sem_t):
        wid = lax.axis_index("s") * NC + lax.axis_index("c")
        base = wid * b_per_w
        pltpu.sync_copy(heads_hbm.at[pl.ds(base, b_per_w)], hidx)
        pltpu.sync_copy(tails_hbm.at[pl.ds(base, b_per_w)], tidx)
        pltpu.sync_copy(rels_hbm.at[pl.ds(base, b_per_w)], ridx)
        lane = lax.iota(jnp.int32, L)

        @pl.loop(0, n_chunks)
        def _chunk(ci):
            off = ci * CH
            cp_r = pltpu.async_copy(
                rel_hbm.at[ridx.at[pl.ds(off, CH)]], rrow, sem_r)

            @pl.loop(0, CH // L)
            def _fire(g):
                vh = hidx[pl.ds(off + g * L, L)]
                vt = tidx[pl.ds(off + g * L, L)]
                for k in range(L):
                    i = g * L + k
                    pltpu.async_copy(
                        ent_hbm.at[lax.shift_right_logical(vh[k], 3)],
                        hslab.at[i], sem_h)
                    pltpu.async_copy(
                        ent_hbm.at[lax.shift_right_logical(vt[k], 3)],
                        tslab.at[i], sem_t)

            pltpu.make_async_copy(ent_hbm.at[pl.ds(0, CH)], hslab,
                                  sem_h).wait()
            pltpu.make_async_copy(ent_hbm.at[pl.ds(0, CH)], tslab,
                                  sem_t).wait()
            cp_r.wait()

            @pl.loop(0, CH // L)
            def _grp(g):
                vh = hidx[pl.ds(off + g * L, L)] & 7
                vt = tidx[pl.ds(off + g * L, L)] & 7
                vec = jnp.zeros((L,), jnp.float32)
                for k in range(L):
                    i = g * L + k
                    hs = vh[k]
                    ts = vt[k]
                    acc = None
                    for c in range(D // L):
                        sl = pl.ds(c * L, L)
                        d = (hslab[i, hs, sl] + rrow[i, sl]
                             - tslab[i, ts, sl])
                        sq = d * d
                        acc = sq if acc is None else acc + sq
                    s = jnp.sum(acc)
                    vec = jnp.where(lane == k, s, vec)
                outc[pl.ds(g * L, L)] = _sqrt16(vec)

            pltpu.sync_copy(outc, out_hbm.at[pl.ds(base + off, CH)])

    return sc_kernel(heads, relations, tails, ent3, relp)


# speedup vs baseline: 2.1558x; 1.0363x over previous
import functools

import jax
import jax.numpy as jnp
from jax import lax
from jax.experimental import pallas as pl
from jax.experimental.pallas import tpu as pltpu
from jax.experimental.pallas import tpu_sc as plsc

L = 16
NC, NS = 2, 16
NW = NC * NS
CH = 16


def _sqrt16(x):
    xs = jnp.maximum(x, jnp.float32(1e-30))
    i = lax.bitcast_convert_type(xs, jnp.int32)
    i = jnp.int32(0x5F3759DF) - lax.shift_right_logical(i, jnp.int32(1))
    y = lax.bitcast_convert_type(i, jnp.float32)
    xh = xs * jnp.float32(0.5)
    for _ in range(3):
        y = y * (jnp.float32(1.5) - xh * y * y)
    return xs * y


def kernel(heads, relations, tails, entity_emb, relation_emb):
    B = heads.shape[0]
    N, D = entity_emb.shape
    b_per_w = B // NW
    n_chunks = b_per_w // CH
    ent3 = entity_emb.reshape(N // 8, 8, D)
    relp = jnp.pad(relation_emb, ((0, 0), (0, 128 - D)))
    mesh = plsc.VectorSubcoreMesh(core_axis_name="c", subcore_axis_name="s")

    @functools.partial(
        pl.kernel,
        mesh=mesh,
        compiler_params=pltpu.CompilerParams(needs_layout_passes=False),
        out_type=jax.ShapeDtypeStruct((B,), jnp.float32),
        scratch_types=[
            pltpu.VMEM((b_per_w,), jnp.int32),
            pltpu.VMEM((b_per_w,), jnp.int32),
            pltpu.VMEM((b_per_w,), jnp.int32),
            pltpu.VMEM((2, CH, 8, D), jnp.float32),
            pltpu.VMEM((2, CH, 8, D), jnp.float32),
            pltpu.VMEM((2, CH, 128), jnp.float32),
            pltpu.VMEM((CH,), jnp.float32),
            pltpu.SemaphoreType.DMA((2,)),
            pltpu.SemaphoreType.DMA((2,)),
            pltpu.SemaphoreType.DMA((2,)),
        ],
    )
    def sc_kernel(heads_hbm, rels_hbm, tails_hbm, ent_hbm, rel_hbm, out_hbm,
                  hidx, tidx, ridx, hslab, tslab, rrow, outc,
                  sem_h, sem_r, sem_t):
        wid = lax.axis_index("s") * NC + lax.axis_index("c")
        base = wid * b_per_w
        pltpu.sync_copy(heads_hbm.at[pl.ds(base, b_per_w)], hidx)
        pltpu.sync_copy(tails_hbm.at[pl.ds(base, b_per_w)], tidx)
        pltpu.sync_copy(rels_hbm.at[pl.ds(base, b_per_w)], ridx)
        lane = lax.iota(jnp.int32, L)

        def fire(off, slot):
            pltpu.async_copy(
                rel_hbm.at[ridx.at[pl.ds(off, CH)]], rrow.at[slot],
                sem_r.at[slot])
            vh = hidx[pl.ds(off, CH)]
            vt = tidx[pl.ds(off, CH)]
            for k in range(CH):
                pltpu.async_copy(
                    ent_hbm.at[lax.shift_right_logical(vh[k], 3)],
                    hslab.at[slot, k], sem_h.at[slot])
                pltpu.async_copy(
                    ent_hbm.at[lax.shift_right_logical(vt[k], 3)],
                    tslab.at[slot, k], sem_t.at[slot])

        def wait(slot):
            pltpu.make_async_copy(ent_hbm.at[pl.ds(0, CH)],
                                  hslab.at[slot], sem_h.at[slot]).wait()
            pltpu.make_async_copy(ent_hbm.at[pl.ds(0, CH)],
                                  tslab.at[slot], sem_t.at[slot]).wait()
            pltpu.make_async_copy(rel_hbm.at[pl.ds(0, CH)],
                                  rrow.at[slot], sem_r.at[slot]).wait()

        def compute(off, slot):
            vh = hidx[pl.ds(off, CH)] & 7
            vt = tidx[pl.ds(off, CH)] & 7
            vec = jnp.zeros((L,), jnp.float32)
            for k in range(CH):
                hs = vh[k]
                ts = vt[k]
                acc = None
                for c in range(D // L):
                    sl = pl.ds(c * L, L)
                    d = (hslab[slot, k, hs, sl] + rrow[slot, k, sl]
                         - tslab[slot, k, ts, sl])
                    sq = d * d
                    acc = sq if acc is None else acc + sq
                s = jnp.sum(acc)
                vec = jnp.where(lane == k, s, vec)
            outc[...] = _sqrt16(vec)
            pltpu.sync_copy(outc, out_hbm.at[pl.ds(base + off, CH)])

        fire(0, 0)

        @pl.loop(0, n_chunks // 2)
        def _pair(p):
            off0 = p * (2 * CH)

            fire(off0 + CH, 1)
            wait(0)
            compute(off0, 0)

            @pl.when(off0 + 2 * CH < b_per_w)
            def _():
                fire(off0 + 2 * CH, 0)

            wait(1)
            compute(off0 + CH, 1)

    return sc_kernel(heads, relations, tails, ent3, relp)


# speedup vs baseline: 2.4580x; 1.1402x over previous
import functools

import jax
import jax.numpy as jnp
from jax import lax
from jax.experimental import pallas as pl
from jax.experimental.pallas import tpu as pltpu
from jax.experimental.pallas import tpu_sc as plsc

L = 16
NC, NS = 2, 16
NW = NC * NS
CH = 16


def _sqrt16(x):
    xs = jnp.maximum(x, jnp.float32(1e-30))
    i = lax.bitcast_convert_type(xs, jnp.int32)
    i = jnp.int32(0x5F3759DF) - lax.shift_right_logical(i, jnp.int32(1))
    y = lax.bitcast_convert_type(i, jnp.float32)
    xh = xs * jnp.float32(0.5)
    for _ in range(3):
        y = y * (jnp.float32(1.5) - xh * y * y)
    return xs * y


def kernel(heads, relations, tails, entity_emb, relation_emb):
    B = heads.shape[0]
    N, D = entity_emb.shape
    b_per_w = B // NW
    n_chunks = b_per_w // CH
    ent3 = entity_emb.reshape(N // 8, 8, D)
    relp = jnp.pad(relation_emb, ((0, 0), (0, 128 - D)))
    mesh = plsc.VectorSubcoreMesh(core_axis_name="c", subcore_axis_name="s")

    @functools.partial(
        pl.kernel,
        mesh=mesh,
        compiler_params=pltpu.CompilerParams(needs_layout_passes=False),
        out_type=jax.ShapeDtypeStruct((B,), jnp.float32),
        scratch_types=[
            pltpu.VMEM((b_per_w,), jnp.int32),
            pltpu.VMEM((b_per_w,), jnp.int32),
            pltpu.VMEM((b_per_w,), jnp.int32),
            pltpu.VMEM((2, CH, D), jnp.float32),
            pltpu.VMEM((2, CH, D), jnp.float32),
            pltpu.VMEM((2, CH, 128), jnp.float32),
            pltpu.VMEM((CH,), jnp.float32),
            pltpu.SemaphoreType.DMA((2,)),
            pltpu.SemaphoreType.DMA((2,)),
            pltpu.SemaphoreType.DMA((2,)),
        ],
    )
    def sc_kernel(heads_hbm, rels_hbm, tails_hbm, ent_hbm, rel_hbm, out_hbm,
                  hidx, tidx, ridx, hslab, tslab, rrow, outc,
                  sem_h, sem_r, sem_t):
        wid = lax.axis_index("s") * NC + lax.axis_index("c")
        base = wid * b_per_w
        pltpu.sync_copy(heads_hbm.at[pl.ds(base, b_per_w)], hidx)
        pltpu.sync_copy(tails_hbm.at[pl.ds(base, b_per_w)], tidx)
        pltpu.sync_copy(rels_hbm.at[pl.ds(base, b_per_w)], ridx)
        lane = lax.iota(jnp.int32, L)

        def fire(off, slot):
            pltpu.async_copy(
                rel_hbm.at[ridx.at[pl.ds(off, CH)]], rrow.at[slot],
                sem_r.at[slot])
            vh = hidx[pl.ds(off, CH)]
            vt = tidx[pl.ds(off, CH)]
            for k in range(CH):
                pltpu.async_copy(
                    ent_hbm.at[lax.shift_right_logical(vh[k], 3), vh[k] & 7],
                    hslab.at[slot, k], sem_h.at[slot])
                pltpu.async_copy(
                    ent_hbm.at[lax.shift_right_logical(vt[k], 3), vt[k] & 7],
                    tslab.at[slot, k], sem_t.at[slot])

        def wait(slot):
            pltpu.make_async_copy(ent_hbm.at[pl.ds(0, CH), 0],
                                  hslab.at[slot], sem_h.at[slot]).wait()
            pltpu.make_async_copy(ent_hbm.at[pl.ds(0, CH), 0],
                                  tslab.at[slot], sem_t.at[slot]).wait()
            pltpu.make_async_copy(rel_hbm.at[pl.ds(0, CH)],
                                  rrow.at[slot], sem_r.at[slot]).wait()

        def compute(off, slot):
            vec = jnp.zeros((L,), jnp.float32)
            for k in range(CH):
                acc = None
                for c in range(D // L):
                    sl = pl.ds(c * L, L)
                    d = (hslab[slot, k, sl] + rrow[slot, k, sl]
                         - tslab[slot, k, sl])
                    sq = d * d
                    acc = sq if acc is None else acc + sq
                s = jnp.sum(acc)
                vec = jnp.where(lane == k, s, vec)
            outc[...] = _sqrt16(vec)
            pltpu.sync_copy(outc, out_hbm.at[pl.ds(base + off, CH)])

        fire(0, 0)

        @pl.loop(0, n_chunks // 2)
        def _pair(p):
            off0 = p * (2 * CH)

            fire(off0 + CH, 1)
            wait(0)
            compute(off0, 0)

            @pl.when(off0 + 2 * CH < b_per_w)
            def _():
                fire(off0 + 2 * CH, 0)

            wait(1)
            compute(off0 + CH, 1)

    return sc_kernel(heads, relations, tails, ent3, relp)
